# Initial kernel scaffold; baseline (speedup 1.0000x reference)
import jax
import jax.numpy as jnp
from jax.experimental import pallas as pl


def kernel(train_data, pre_phrase, position_number, W_enc, b_enc, W_dec, b_dec, phrase_table, codebook):
    raise NotImplementedError("write your pallas kernel here")


# speedup vs baseline: 2.9173x; 2.9173x over previous
import jax
import jax.numpy as jnp
from jax.experimental import pallas as pl

_B = 1024
_IN = 4096
_D = 510
_DP = 512
_K = 128
_P = 332
_PP = 336
_BBLK = 128
_GRID = _B // _BBLK
_HI = jax.lax.Precision.HIGHEST


def _shift_left(x, j):
    pad = jnp.zeros((x.shape[0], j), jnp.float32)
    return jnp.concatenate([x[:, j:], pad], axis=1)


def _ref_dist(z, c):
    diff = z - c
    sq = diff * diff
    s = sq
    for j in range(1, 6):
        s = s + _shift_left(sq, j)
    st = jnp.transpose(s)
    acc = st[0:1, :]
    for g in range(1, 85):
        acc = acc + st[6 * g:6 * g + 1, :]
    return jnp.sqrt(acc)


def _cands(zz, C, CT, cb_sq):
    kiota = jax.lax.broadcasted_iota(jnp.int32, (zz.shape[0], _K), 1)
    s = cb_sq - 2.0 * jnp.dot(zz, CT, preferred_element_type=jnp.float32)
    i1 = jnp.argmin(s, axis=-1)
    s2 = jnp.where(kiota == i1[:, None], jnp.float32(jnp.inf), s)
    i2 = jnp.argmin(s2, axis=-1)
    oh1 = (kiota == i1[:, None]).astype(jnp.float32)
    oh2 = (kiota == i2[:, None]).astype(jnp.float32)
    c1 = jnp.dot(oh1, C, precision=_HI, preferred_element_type=jnp.float32)
    c2 = jnp.dot(oh2, C, precision=_HI, preferred_element_type=jnp.float32)
    return i1, i2, c1, c2


def _fused(pos_ref, train_ref, pre_ref, we_ref, be_ref, wd_ref, bd_ref,
           ph_ref, cb_ref, cbt_ref, cbsq_ref, ori_ref, music_ref, z_ref,
           zq_ref, zp_ref, zpq_ref):
    C = cb_ref[...]
    CT = cbt_ref[...]
    cb_sq = cbsq_ref[...]
    we = we_ref[...]
    be = be_ref[...]
    z = jnp.tanh(jnp.dot(train_ref[...], we,
                         preferred_element_type=jnp.float32) + be)
    zp = jnp.tanh(jnp.dot(pre_ref[...], we,
                          preferred_element_type=jnp.float32) + be)
    zz = jnp.concatenate([z, zp], axis=0)
    i1, i2, c1, c2 = _cands(zz, C, CT, cb_sq)
    big = jnp.concatenate([zz, zz], axis=0)
    cc = jnp.concatenate([c1, c2], axis=0)
    d = _ref_dist(big, cc)
    d1 = d[:, 0:2 * _BBLK]
    d2 = d[:, 2 * _BBLK:]
    take2 = (d2 < d1) | ((d2 == d1) & (i2 < i1)[None, :])
    zqq = jnp.where(jnp.transpose(take2), c2, c1)
    zq = zqq[0:_BBLK]
    zpq = zqq[_BBLK:]
    pos = pos_ref[0, 0, :]
    piota = jax.lax.broadcasted_iota(jnp.int32, (_BBLK, _PP), 1)
    poh = (piota == pos[:, None]).astype(jnp.float32)
    posv = jnp.dot(poh, ph_ref[...], precision=_HI,
                   preferred_element_type=jnp.float32)
    logits = jnp.dot(zq + zpq + posv, wd_ref[...],
                     preferred_element_type=jnp.float32) + bd_ref[...]
    ori = jax.nn.sigmoid(logits)
    ori_ref[...] = ori
    music_ref[...] = (ori > 0.35).astype(jnp.float32)
    z_ref[...] = z
    zq_ref[...] = zq
    zp_ref[...] = zp
    zpq_ref[...] = zpq


def kernel(train_data, pre_phrase, position_number, W_enc, b_enc, W_dec,
           b_dec, phrase_table, codebook):
    f32 = jnp.float32
    we = jnp.pad(W_enc, ((0, 0), (0, _DP - _D)))
    be = jnp.pad(b_enc, (0, _DP - _D)).reshape(1, _DP)
    wd = jnp.pad(W_dec, ((0, _DP - _D), (0, 0)))
    bd = b_dec.reshape(1, _IN)
    ph = jnp.pad(phrase_table, ((0, _PP - _P), (0, _DP - _D)))
    cb = jnp.pad(codebook, ((0, 0), (0, _DP - _D)))
    cbt = cb.T
    cbsq = jnp.sum(cb * cb, axis=1).reshape(1, _K)
    pos = position_number.astype(jnp.int32).reshape(_GRID, 1, _BBLK)

    blk_b = lambda i: (i, 0)
    const2 = lambda i: (0, 0)
    outs = pl.pallas_call(
        _fused,
        grid=(_GRID,),
        in_specs=[
            pl.BlockSpec((1, 1, _BBLK), lambda i: (i, 0, 0)),
            pl.BlockSpec((_BBLK, _IN), blk_b),
            pl.BlockSpec((_BBLK, _IN), blk_b),
            pl.BlockSpec((_IN, _DP), const2),
            pl.BlockSpec((1, _DP), const2),
            pl.BlockSpec((_DP, _IN), const2),
            pl.BlockSpec((1, _IN), const2),
            pl.BlockSpec((_PP, _DP), const2),
            pl.BlockSpec((_K, _DP), const2),
            pl.BlockSpec((_DP, _K), const2),
            pl.BlockSpec((1, _K), const2),
        ],
        out_specs=[
            pl.BlockSpec((_BBLK, _IN), blk_b),
            pl.BlockSpec((_BBLK, _IN), blk_b),
            pl.BlockSpec((_BBLK, _DP), blk_b),
            pl.BlockSpec((_BBLK, _DP), blk_b),
            pl.BlockSpec((_BBLK, _DP), blk_b),
            pl.BlockSpec((_BBLK, _DP), blk_b),
        ],
        out_shape=[
            jax.ShapeDtypeStruct((_B, _IN), f32),
            jax.ShapeDtypeStruct((_B, _IN), f32),
            jax.ShapeDtypeStruct((_B, _DP), f32),
            jax.ShapeDtypeStruct((_B, _DP), f32),
            jax.ShapeDtypeStruct((_B, _DP), f32),
            jax.ShapeDtypeStruct((_B, _DP), f32),
        ],
    )(pos, train_data, pre_phrase, we, be, wd, bd, ph, cb, cbt, cbsq)
    ori, music, z, zq, zp, zpq = outs
    return (ori, music, z[:, :_D], zq[:, :_D], zp[:, :_D], zpq[:, :_D])
